# baseline (speedup 1.0000x reference)
import functools

import jax
import jax.numpy as jnp
import numpy as np
from jax import lax
from jax.experimental import pallas as pl
from jax.experimental.pallas import tpu as pltpu
from jax.experimental.pallas import tpu_sc as plsc

N = 10000
E = 320000
D = 128
H = 8
DH = 16
HC = H // 2
ACC_W = 72

C = 64
IW = 128
EROWS = E // IW
SLAB = 160
RPT = 1000

ROWS = 1000
GRID = N // ROWS

_NCH = [157 if t < 4 else 156 for t in range(16)]
_RS = [157 * t if t < 4 else 628 + 156 * (t - 4) for t in range(16)]
_ROWMAP = np.array(
    [[_RS[t] + r if r < _NCH[t] else EROWS for r in range(SLAB)] for t in range(16)],
    dtype=np.int32,
)


def _qkv_body(h_ref, wq_ref, wk_ref, wv_ref, q_ref, kvh_ref):
    x = h_ref[...]
    q = jnp.dot(x, wq_ref[...], preferred_element_type=jnp.float32)
    k = jnp.dot(x, wk_ref[...], preferred_element_type=jnp.float32)
    v = jnp.dot(x, wv_ref[...], preferred_element_type=jnp.float32)
    half = HC * DH
    q_ref[...] = q
    kvh_ref[0] = jnp.concatenate([k[:, :half], v[:, :half]], axis=1)
    kvh_ref[1] = jnp.concatenate([k[:, half:], v[:, half:]], axis=1)


def _qkv(h, WQ, WK, WV):
    return pl.pallas_call(
        _qkv_body,
        grid=(GRID,),
        in_specs=[
            pl.BlockSpec((ROWS, D), lambda i: (i, 0)),
            pl.BlockSpec((D, D), lambda i: (0, 0)),
            pl.BlockSpec((D, D), lambda i: (0, 0)),
            pl.BlockSpec((D, D), lambda i: (0, 0)),
        ],
        out_specs=[
            pl.BlockSpec((ROWS, D), lambda i: (i, 0)),
            pl.BlockSpec((2, ROWS, D), lambda i: (0, i, 0)),
        ],
        out_shape=[
            jax.ShapeDtypeStruct((N, D), jnp.float32),
            jax.ShapeDtypeStruct((2, N, D), jnp.float32),
        ],
    )(h, WQ, WK, WV)


def _edge_body(srcg_ref, dst_ref, q_ref, kvh_ref, out_ref,
               src_t, dst_t, dl0, dl1, kv0, kv1, q0, q1, out0, out1, acc_sh,
               sem_kv0, sem_kv1, sem_q0, sem_q1, sem_sc0, sem_sc1):
    cid = lax.axis_index("c")
    sid = lax.axis_index("s")

    nch = jnp.where(sid < 4, 2 * 157, 2 * 156)

    pltpu.sync_copy(srcg_ref.at[cid * 16 + sid], src_t)
    pltpu.sync_copy(dst_ref.at[sid], dst_t)

    @pl.loop(0, C)
    def _(ee):
        for ob in (out0, out1):
            for j in range(4):
                ob[ee, pl.ds(j * 16, 16)] = jnp.zeros((16,), jnp.float32)
            ob[ee, pl.ds(ACC_W - 16, 16)] = jnp.zeros((16,), jnp.float32)

    @pl.when(sid < N // RPT)
    def _():
        @pl.loop(0, RPT // 40)
        def _(j):
            pltpu.sync_copy(out0.at[pl.ds(0, 40)],
                            acc_sh.at[pl.ds(sid * RPT + j * 40, 40)])

    plsc.subcore_barrier()

    kvbufs = (kv0, kv1)
    qbufs = (q0, q1)
    semkv = (sem_kv0, sem_kv1)
    semq = (sem_q0, sem_q1)
    outbufs = (out0, out1)
    dlbufs = (dl0, dl1)
    semsc = (sem_sc0, sem_sc1)

    def issue(g, b):
        r = g // 2
        hf = (g % 2) * C
        pltpu.async_copy(kvh_ref.at[src_t.at[r, pl.ds(hf, C)]], kvbufs[b], semkv[b])
        pltpu.async_copy(q_ref.at[dst_t.at[r, pl.ds(hf, C)]], qbufs[b], semq[b])

    def wait(b):
        pltpu.make_async_copy(kvh_ref.at[src_t.at[0, pl.ds(0, C)]], kvbufs[b], semkv[b]).wait()
        pltpu.make_async_copy(q_ref.at[dst_t.at[0, pl.ds(0, C)]], qbufs[b], semq[b]).wait()

    def compute(g, b):
        kvb = kvbufs[b]
        qb = qbufs[b]
        out_b = outbufs[b]
        dstl_row = dlbufs[b]

        @pl.when(g >= 2)
        def _():
            pltpu.make_async_copy(out_b, acc_sh.at[dstl_row], semsc[b]).wait()

        @pl.loop(0, C // 16)
        def _(e16):
            eidx = e16 * 16 + lax.broadcasted_iota(jnp.int32, (16,), 0)
            for hh in range(HC):
                col = jnp.full((16,), hh * DH, jnp.int32)
                qcol = col + cid * (HC * DH)
                a0 = jnp.zeros((16,), jnp.float32)
                a1 = jnp.zeros((16,), jnp.float32)
                a2 = jnp.zeros((16,), jnp.float32)
                a3 = jnp.zeros((16,), jnp.float32)
                for dd in range(DH):
                    kvals = plsc.load_gather(kvb, [eidx, col + dd])
                    qvals = plsc.load_gather(qb, [eidx, qcol + dd])
                    prod = kvals * qvals
                    if dd % 4 == 0:
                        a0 = a0 + prod
                    elif dd % 4 == 1:
                        a1 = a1 + prod
                    elif dd % 4 == 2:
                        a2 = a2 + prod
                    else:
                        a3 = a3 + prod
                s = (a0 + a1) + (a2 + a3)
                w = jnp.exp(jnp.clip(s * 0.25, -5.0, 5.0))
                plsc.store_scatter(out_b, [eidx, jnp.full((16,), HC * DH + hh, jnp.int32)], w)
                for dd in range(DH):
                    vvals = plsc.load_gather(kvb, [eidx, col + (HC * DH + dd)])
                    plsc.store_scatter(out_b, [eidx, col + dd], w * vvals)

        r = g // 2
        hf = (g % 2) * C

        @pl.loop(0, C // 16)
        def _(jj):
            dstl_row[pl.ds(jj * 16, 16)] = dst_t[r, pl.ds(hf + jj * 16, 16)]

        pltpu.async_copy(out_b, acc_sh.at[dstl_row], semsc[b], add=True)

    issue(0, 0)

    @pl.loop(0, 314, step=2)
    def _(g):
        for b in range(2):
            gi = g + b

            @pl.when(gi < nch)
            def _():
                @pl.when(gi + 1 < nch)
                def _():
                    issue(gi + 1, 1 - b)

                wait(b)
                compute(gi, b)

    pltpu.make_async_copy(out0, acc_sh.at[dl0], sem_sc0).wait()
    pltpu.make_async_copy(out1, acc_sh.at[dl1], sem_sc1).wait()

    plsc.subcore_barrier()

    @pl.when(sid < N // RPT)
    def _():
        @pl.loop(0, RPT // 40)
        def _(j):
            pltpu.sync_copy(acc_sh.at[pl.ds(sid * RPT + j * 40, 40)],
                            out_ref.at[pl.ds(cid * N + sid * RPT + j * 40, 40),
                                       pl.ds(0, ACC_W)])


@functools.partial(
    pl.kernel,
    out_type=jax.ShapeDtypeStruct((2 * N, D), jnp.float32),
    mesh=plsc.VectorSubcoreMesh(core_axis_name="c", subcore_axis_name="s"),
    compiler_params=pltpu.CompilerParams(use_tc_tiling_on_sc=False, needs_layout_passes=False),
    scratch_types=[
        pltpu.VMEM((SLAB, IW), jnp.int32),
        pltpu.VMEM((SLAB, IW), jnp.int32),
        pltpu.VMEM((C,), jnp.int32),
        pltpu.VMEM((C,), jnp.int32),
        pltpu.VMEM((C, D), jnp.float32),
        pltpu.VMEM((C, D), jnp.float32),
        pltpu.VMEM((C, D), jnp.float32),
        pltpu.VMEM((C, D), jnp.float32),
        pltpu.VMEM((C, ACC_W), jnp.float32),
        pltpu.VMEM((C, ACC_W), jnp.float32),
        pltpu.VMEM_SHARED((N, ACC_W), jnp.float32),
        pltpu.SemaphoreType.DMA,
        pltpu.SemaphoreType.DMA,
        pltpu.SemaphoreType.DMA,
        pltpu.SemaphoreType.DMA,
        pltpu.SemaphoreType.DMA,
        pltpu.SemaphoreType.DMA,
    ],
)
def _edge_sc(srcg_ref, dst_ref, q_ref, kvh_ref, out_ref,
             src_t, dst_t, dl0, dl1, kv0, kv1, q0, q1, out0, out1, acc_sh,
             sem_kv0, sem_kv1, sem_q0, sem_q1, sem_sc0, sem_sc1):
    _edge_body(srcg_ref, dst_ref, q_ref, kvh_ref, out_ref,
               src_t, dst_t, dl0, dl1, kv0, kv1, q0, q1, out0, out1, acc_sh,
               sem_kv0, sem_kv1, sem_q0, sem_q1, sem_sc0, sem_sc1)


def _attnout_body(acca_ref, accb_ref, h_ref, wo_ref, bo_ref, h1_ref, stats_ref):
    acca = acca_ref[...]
    accb = accb_ref[...]
    zcol = HC * DH
    z = jnp.concatenate([acca[:, zcol:zcol + HC], accb[:, zcol:zcol + HC]], axis=1)
    zr = 1.0 / (z + 1e-6)
    hsel = lax.broadcasted_iota(jnp.int32, (H, D), 0)
    dsel = lax.broadcasted_iota(jnp.int32, (H, D), 1) // DH
    expand = (hsel == dsel).astype(jnp.float32)
    zfull = jnp.dot(zr, expand, preferred_element_type=jnp.float32)
    accv = jnp.concatenate([acca[:, :zcol], accb[:, :zcol]], axis=1)
    h_attn = accv * zfull
    h_out = jnp.dot(h_attn, wo_ref[...], preferred_element_type=jnp.float32)
    h1 = h_ref[...] + h_out + bo_ref[...]
    h1_ref[...] = h1
    s = jnp.concatenate(
        [jnp.sum(h1, axis=0, keepdims=True), jnp.sum(h1 * h1, axis=0, keepdims=True)], axis=0
    )

    @pl.when(pl.program_id(0) == 0)
    def _():
        stats_ref[...] = jnp.zeros_like(stats_ref)

    stats_ref[...] += s


def _attnout(acc2, h, WO, bO):
    return pl.pallas_call(
        _attnout_body,
        grid=(GRID,),
        in_specs=[
            pl.BlockSpec((ROWS, D), lambda i: (i, 0)),
            pl.BlockSpec((ROWS, D), lambda i: (GRID + i, 0)),
            pl.BlockSpec((ROWS, D), lambda i: (i, 0)),
            pl.BlockSpec((D, D), lambda i: (0, 0)),
            pl.BlockSpec((1, D), lambda i: (0, 0)),
        ],
        out_specs=[
            pl.BlockSpec((ROWS, D), lambda i: (i, 0)),
            pl.BlockSpec((2, D), lambda i: (0, 0)),
        ],
        out_shape=[
            jax.ShapeDtypeStruct((N, D), jnp.float32),
            jax.ShapeDtypeStruct((2, D), jnp.float32),
        ],
    )(acc2, acc2, h, WO, bO.reshape(1, D))


def _bn_ffn_body(h1_ref, stats_ref, g_ref, b_ref, w1_ref, b1_ref, w2_ref, b2_ref,
                 h3_ref, stats2_ref):
    s = stats_ref[...]
    mu = s[0:1, :] / N
    var = s[1:2, :] / N - mu * mu
    scale = g_ref[...] * lax.rsqrt(var + 1e-5)
    hn = (h1_ref[...] - mu) * scale + b_ref[...]
    a = jnp.dot(hn, w1_ref[...], preferred_element_type=jnp.float32) + b1_ref[...]
    a = jnp.maximum(a, 0.0)
    h2 = jnp.dot(a, w2_ref[...], preferred_element_type=jnp.float32) + b2_ref[...]
    h3 = hn + h2
    h3_ref[...] = h3
    s2 = jnp.concatenate(
        [jnp.sum(h3, axis=0, keepdims=True), jnp.sum(h3 * h3, axis=0, keepdims=True)], axis=0
    )

    @pl.when(pl.program_id(0) == 0)
    def _():
        stats2_ref[...] = jnp.zeros_like(stats2_ref)

    stats2_ref[...] += s2


def _bn_ffn(h1, stats, bn1_g, bn1_b, W1, b1, W2, b2):
    return pl.pallas_call(
        _bn_ffn_body,
        grid=(GRID,),
        in_specs=[
            pl.BlockSpec((ROWS, D), lambda i: (i, 0)),
            pl.BlockSpec((2, D), lambda i: (0, 0)),
            pl.BlockSpec((1, D), lambda i: (0, 0)),
            pl.BlockSpec((1, D), lambda i: (0, 0)),
            pl.BlockSpec((D, 2 * D), lambda i: (0, 0)),
            pl.BlockSpec((1, 2 * D), lambda i: (0, 0)),
            pl.BlockSpec((2 * D, D), lambda i: (0, 0)),
            pl.BlockSpec((1, D), lambda i: (0, 0)),
        ],
        out_specs=[
            pl.BlockSpec((ROWS, D), lambda i: (i, 0)),
            pl.BlockSpec((2, D), lambda i: (0, 0)),
        ],
        out_shape=[
            jax.ShapeDtypeStruct((N, D), jnp.float32),
            jax.ShapeDtypeStruct((2, D), jnp.float32),
        ],
    )(h1, stats, bn1_g.reshape(1, D), bn1_b.reshape(1, D), W1, b1.reshape(1, 2 * D),
      W2, b2.reshape(1, D))


def _bn2_body(h3_ref, stats_ref, g_ref, b_ref, out_ref):
    s = stats_ref[...]
    mu = s[0:1, :] / N
    var = s[1:2, :] / N - mu * mu
    scale = g_ref[...] * lax.rsqrt(var + 1e-5)
    out_ref[...] = (h3_ref[...] - mu) * scale + b_ref[...]


def _bn2(h3, stats2, bn2_g, bn2_b):
    return pl.pallas_call(
        _bn2_body,
        grid=(GRID,),
        in_specs=[
            pl.BlockSpec((ROWS, D), lambda i: (i, 0)),
            pl.BlockSpec((2, D), lambda i: (0, 0)),
            pl.BlockSpec((1, D), lambda i: (0, 0)),
            pl.BlockSpec((1, D), lambda i: (0, 0)),
        ],
        out_specs=pl.BlockSpec((ROWS, D), lambda i: (i, 0)),
        out_shape=jax.ShapeDtypeStruct((N, D), jnp.float32),
    )(h3, stats2, bn2_g.reshape(1, D), bn2_b.reshape(1, D))


def kernel(h, p, e, edge_index, WQ, WK, WE, WV, WO, bO, bn1_g, bn1_b, W1, b1, W2, b2,
           bn2_g, bn2_b):
    q, kvh3 = _qkv(h, WQ, WK, WV)
    kvh2 = kvh3.reshape(2 * N, D)
    rowmap = jnp.asarray(_ROWMAP)
    src2d = jnp.concatenate(
        [edge_index[0].reshape(EROWS, IW), jnp.zeros((1, IW), jnp.int32)], axis=0
    )
    dst2d = jnp.concatenate(
        [edge_index[1].reshape(EROWS, IW), jnp.zeros((1, IW), jnp.int32)], axis=0
    )
    src_slabs = jnp.take(src2d, rowmap, axis=0)
    dst_slabs = jnp.take(dst2d, rowmap, axis=0)
    srcg = jnp.concatenate([src_slabs, src_slabs + N], axis=0)
    acc2 = _edge_sc(srcg, dst_slabs, q, kvh2)
    h1, stats = _attnout(acc2, h, WO, bO)
    h3, stats2 = _bn_ffn(h1, stats, bn1_g, bn1_b, W1, b1, W2, b2)
    return _bn2(h3, stats2, bn2_g, bn2_b)


# speedup vs baseline: 9.1236x; 9.1236x over previous
import functools

import jax
import jax.numpy as jnp
import numpy as np
from jax import lax
from jax.experimental import pallas as pl
from jax.experimental.pallas import tpu as pltpu
from jax.experimental.pallas import tpu_sc as plsc

N = 10000
E = 320000
D = 128
H = 8
DH = 16
HC = H // 2
ACC_W = 72

C = 64
IW = 128
EROWS = E // IW
SLAB = 160
RPT = 1000

ROWS = 1000
GRID = N // ROWS

_NCH = [157 if t < 4 else 156 for t in range(16)]
_RS = [157 * t if t < 4 else 628 + 156 * (t - 4) for t in range(16)]
_ROWMAP = np.array(
    [[_RS[t] + r if r < _NCH[t] else EROWS for r in range(SLAB)] for t in range(16)],
    dtype=np.int32,
)


def _qkv_body(h_ref, wq_ref, wk_ref, wv_ref, q_ref, kvh_ref):
    x = h_ref[...]
    q = jnp.dot(x, wq_ref[...], preferred_element_type=jnp.float32)
    k = jnp.dot(x, wk_ref[...], preferred_element_type=jnp.float32)
    v = jnp.dot(x, wv_ref[...], preferred_element_type=jnp.float32)
    half = HC * DH
    q_ref[...] = q
    kvh_ref[0] = jnp.concatenate([k[:, :half], v[:, :half]], axis=1)
    kvh_ref[1] = jnp.concatenate([k[:, half:], v[:, half:]], axis=1)


def _qkv(h, WQ, WK, WV):
    return pl.pallas_call(
        _qkv_body,
        grid=(GRID,),
        in_specs=[
            pl.BlockSpec((ROWS, D), lambda i: (i, 0)),
            pl.BlockSpec((D, D), lambda i: (0, 0)),
            pl.BlockSpec((D, D), lambda i: (0, 0)),
            pl.BlockSpec((D, D), lambda i: (0, 0)),
        ],
        out_specs=[
            pl.BlockSpec((ROWS, D), lambda i: (i, 0)),
            pl.BlockSpec((2, ROWS, D), lambda i: (0, i, 0)),
        ],
        out_shape=[
            jax.ShapeDtypeStruct((N, D), jnp.float32),
            jax.ShapeDtypeStruct((2, N, D), jnp.float32),
        ],
    )(h, WQ, WK, WV)


def _edge_body(srcg_ref, dst_ref, q_ref, kvh_ref, out_ref,
               src_t, dst_t, dstl_row, kv0, kv1, q0, q1, out_b, acc_sh,
               sem_kv0, sem_kv1, sem_q0, sem_q1):
    cid = lax.axis_index("c")
    sid = lax.axis_index("s")

    nch = jnp.where(sid < 4, 2 * 157, 2 * 156)

    pltpu.sync_copy(srcg_ref.at[cid * 16 + sid], src_t)
    pltpu.sync_copy(dst_ref.at[sid], dst_t)

    @pl.loop(0, C)
    def _(ee):
        for j in range(4):
            out_b[ee, pl.ds(j * 16, 16)] = jnp.zeros((16,), jnp.float32)
        out_b[ee, pl.ds(ACC_W - 16, 16)] = jnp.zeros((16,), jnp.float32)

    @pl.when(sid < N // RPT)
    def _():
        @pl.loop(0, RPT // 40)
        def _(j):
            pltpu.sync_copy(out_b.at[pl.ds(0, 40)],
                            acc_sh.at[pl.ds(sid * RPT + j * 40, 40)])

    plsc.subcore_barrier()

    kvbufs = (kv0, kv1)
    qbufs = (q0, q1)
    semkv = (sem_kv0, sem_kv1)
    semq = (sem_q0, sem_q1)

    def issue(g, b):
        r = g // 2
        hf = (g % 2) * C
        pltpu.async_copy(kvh_ref.at[src_t.at[r, pl.ds(hf, C)]], kvbufs[b], semkv[b])
        pltpu.async_copy(q_ref.at[dst_t.at[r, pl.ds(hf, C)]], qbufs[b], semq[b])

    def wait(b):
        pltpu.make_async_copy(kvh_ref.at[src_t.at[0, pl.ds(0, C)]], kvbufs[b], semkv[b]).wait()
        pltpu.make_async_copy(q_ref.at[dst_t.at[0, pl.ds(0, C)]], qbufs[b], semq[b]).wait()

    def compute(g, b):
        kvb = kvbufs[b]
        qb = qbufs[b]

        @pl.loop(0, C // 16)
        def _(e16):
            lane = lax.broadcasted_iota(jnp.int32, (16,), 0)
            eidx = e16 * 16 + lane
            rots = [(lane + t) & 15 for t in range(DH)]
            qoff = cid * (HC * DH)
            for hh in range(HC):
                base = hh * DH
                a0 = jnp.zeros((16,), jnp.float32)
                a1 = jnp.zeros((16,), jnp.float32)
                a2 = jnp.zeros((16,), jnp.float32)
                a3 = jnp.zeros((16,), jnp.float32)
                for t in range(DH):
                    colv = rots[t] + base
                    kvals = plsc.load_gather(kvb, [eidx, colv])
                    qvals = plsc.load_gather(qb, [eidx, colv + qoff])
                    prod = kvals * qvals
                    if t % 4 == 0:
                        a0 = a0 + prod
                    elif t % 4 == 1:
                        a1 = a1 + prod
                    elif t % 4 == 2:
                        a2 = a2 + prod
                    else:
                        a3 = a3 + prod
                s = (a0 + a1) + (a2 + a3)
                w = jnp.exp(jnp.clip(s * 0.25, -5.0, 5.0))
                plsc.store_scatter(out_b, [eidx, jnp.full((16,), HC * DH + hh, jnp.int32)], w)
                for t in range(DH):
                    colv = rots[t] + base
                    vvals = plsc.load_gather(kvb, [eidx, colv + (HC * DH)])
                    plsc.store_scatter(out_b, [eidx, colv], w * vvals)

        r = g // 2
        hf = (g % 2) * C

        @pl.loop(0, C // 16)
        def _(jj):
            dstl_row[pl.ds(jj * 16, 16)] = dst_t[r, pl.ds(hf + jj * 16, 16)]

        pltpu.sync_copy(out_b, acc_sh.at[dstl_row], add=True)

    issue(0, 0)

    @pl.loop(0, 314, step=2)
    def _(g):
        for b in range(2):
            gi = g + b

            @pl.when(gi < nch)
            def _():
                @pl.when(gi + 1 < nch)
                def _():
                    issue(gi + 1, 1 - b)

                wait(b)
                compute(gi, b)

    plsc.subcore_barrier()

    @pl.when(sid < N // RPT)
    def _():
        @pl.loop(0, RPT // 40)
        def _(j):
            pltpu.sync_copy(acc_sh.at[pl.ds(sid * RPT + j * 40, 40)],
                            out_ref.at[pl.ds(cid * N + sid * RPT + j * 40, 40),
                                       pl.ds(0, ACC_W)])


@functools.partial(
    pl.kernel,
    out_type=jax.ShapeDtypeStruct((2 * N, D), jnp.float32),
    mesh=plsc.VectorSubcoreMesh(core_axis_name="c", subcore_axis_name="s"),
    compiler_params=pltpu.CompilerParams(use_tc_tiling_on_sc=False, needs_layout_passes=False),
    scratch_types=[
        pltpu.VMEM((SLAB, IW), jnp.int32),
        pltpu.VMEM((SLAB, IW), jnp.int32),
        pltpu.VMEM((C,), jnp.int32),
        pltpu.VMEM((C, D), jnp.float32),
        pltpu.VMEM((C, D), jnp.float32),
        pltpu.VMEM((C, D), jnp.float32),
        pltpu.VMEM((C, D), jnp.float32),
        pltpu.VMEM((C, ACC_W), jnp.float32),
        pltpu.VMEM_SHARED((N, ACC_W), jnp.float32),
        pltpu.SemaphoreType.DMA,
        pltpu.SemaphoreType.DMA,
        pltpu.SemaphoreType.DMA,
        pltpu.SemaphoreType.DMA,
    ],
)
def _edge_sc(srcg_ref, dst_ref, q_ref, kvh_ref, out_ref,
             src_t, dst_t, dstl_row, kv0, kv1, q0, q1, out_b, acc_sh,
             sem_kv0, sem_kv1, sem_q0, sem_q1):
    _edge_body(srcg_ref, dst_ref, q_ref, kvh_ref, out_ref,
               src_t, dst_t, dstl_row, kv0, kv1, q0, q1, out_b, acc_sh,
               sem_kv0, sem_kv1, sem_q0, sem_q1)


def _attnout_body(acca_ref, accb_ref, h_ref, wo_ref, bo_ref, h1_ref, stats_ref):
    acca = acca_ref[...]
    accb = accb_ref[...]
    zcol = HC * DH
    z = jnp.concatenate([acca[:, zcol:zcol + HC], accb[:, zcol:zcol + HC]], axis=1)
    zr = 1.0 / (z + 1e-6)
    hsel = lax.broadcasted_iota(jnp.int32, (H, D), 0)
    dsel = lax.broadcasted_iota(jnp.int32, (H, D), 1) // DH
    expand = (hsel == dsel).astype(jnp.float32)
    zfull = jnp.dot(zr, expand, preferred_element_type=jnp.float32)
    accv = jnp.concatenate([acca[:, :zcol], accb[:, :zcol]], axis=1)
    h_attn = accv * zfull
    h_out = jnp.dot(h_attn, wo_ref[...], preferred_element_type=jnp.float32)
    h1 = h_ref[...] + h_out + bo_ref[...]
    h1_ref[...] = h1
    s = jnp.concatenate(
        [jnp.sum(h1, axis=0, keepdims=True), jnp.sum(h1 * h1, axis=0, keepdims=True)], axis=0
    )

    @pl.when(pl.program_id(0) == 0)
    def _():
        stats_ref[...] = jnp.zeros_like(stats_ref)

    stats_ref[...] += s


def _attnout(acc2, h, WO, bO):
    return pl.pallas_call(
        _attnout_body,
        grid=(GRID,),
        in_specs=[
            pl.BlockSpec((ROWS, D), lambda i: (i, 0)),
            pl.BlockSpec((ROWS, D), lambda i: (GRID + i, 0)),
            pl.BlockSpec((ROWS, D), lambda i: (i, 0)),
            pl.BlockSpec((D, D), lambda i: (0, 0)),
            pl.BlockSpec((1, D), lambda i: (0, 0)),
        ],
        out_specs=[
            pl.BlockSpec((ROWS, D), lambda i: (i, 0)),
            pl.BlockSpec((2, D), lambda i: (0, 0)),
        ],
        out_shape=[
            jax.ShapeDtypeStruct((N, D), jnp.float32),
            jax.ShapeDtypeStruct((2, D), jnp.float32),
        ],
    )(acc2, acc2, h, WO, bO.reshape(1, D))


def _bn_ffn_body(h1_ref, stats_ref, g_ref, b_ref, w1_ref, b1_ref, w2_ref, b2_ref,
                 h3_ref, stats2_ref):
    s = stats_ref[...]
    mu = s[0:1, :] / N
    var = s[1:2, :] / N - mu * mu
    scale = g_ref[...] * lax.rsqrt(var + 1e-5)
    hn = (h1_ref[...] - mu) * scale + b_ref[...]
    a = jnp.dot(hn, w1_ref[...], preferred_element_type=jnp.float32) + b1_ref[...]
    a = jnp.maximum(a, 0.0)
    h2 = jnp.dot(a, w2_ref[...], preferred_element_type=jnp.float32) + b2_ref[...]
    h3 = hn + h2
    h3_ref[...] = h3
    s2 = jnp.concatenate(
        [jnp.sum(h3, axis=0, keepdims=True), jnp.sum(h3 * h3, axis=0, keepdims=True)], axis=0
    )

    @pl.when(pl.program_id(0) == 0)
    def _():
        stats2_ref[...] = jnp.zeros_like(stats2_ref)

    stats2_ref[...] += s2


def _bn_ffn(h1, stats, bn1_g, bn1_b, W1, b1, W2, b2):
    return pl.pallas_call(
        _bn_ffn_body,
        grid=(GRID,),
        in_specs=[
            pl.BlockSpec((ROWS, D), lambda i: (i, 0)),
            pl.BlockSpec((2, D), lambda i: (0, 0)),
            pl.BlockSpec((1, D), lambda i: (0, 0)),
            pl.BlockSpec((1, D), lambda i: (0, 0)),
            pl.BlockSpec((D, 2 * D), lambda i: (0, 0)),
            pl.BlockSpec((1, 2 * D), lambda i: (0, 0)),
            pl.BlockSpec((2 * D, D), lambda i: (0, 0)),
            pl.BlockSpec((1, D), lambda i: (0, 0)),
        ],
        out_specs=[
            pl.BlockSpec((ROWS, D), lambda i: (i, 0)),
            pl.BlockSpec((2, D), lambda i: (0, 0)),
        ],
        out_shape=[
            jax.ShapeDtypeStruct((N, D), jnp.float32),
            jax.ShapeDtypeStruct((2, D), jnp.float32),
        ],
    )(h1, stats, bn1_g.reshape(1, D), bn1_b.reshape(1, D), W1, b1.reshape(1, 2 * D),
      W2, b2.reshape(1, D))


def _bn2_body(h3_ref, stats_ref, g_ref, b_ref, out_ref):
    s = stats_ref[...]
    mu = s[0:1, :] / N
    var = s[1:2, :] / N - mu * mu
    scale = g_ref[...] * lax.rsqrt(var + 1e-5)
    out_ref[...] = (h3_ref[...] - mu) * scale + b_ref[...]


def _bn2(h3, stats2, bn2_g, bn2_b):
    return pl.pallas_call(
        _bn2_body,
        grid=(GRID,),
        in_specs=[
            pl.BlockSpec((ROWS, D), lambda i: (i, 0)),
            pl.BlockSpec((2, D), lambda i: (0, 0)),
            pl.BlockSpec((1, D), lambda i: (0, 0)),
            pl.BlockSpec((1, D), lambda i: (0, 0)),
        ],
        out_specs=pl.BlockSpec((ROWS, D), lambda i: (i, 0)),
        out_shape=jax.ShapeDtypeStruct((N, D), jnp.float32),
    )(h3, stats2, bn2_g.reshape(1, D), bn2_b.reshape(1, D))


def kernel(h, p, e, edge_index, WQ, WK, WE, WV, WO, bO, bn1_g, bn1_b, W1, b1, W2, b2,
           bn2_g, bn2_b):
    q, kvh3 = _qkv(h, WQ, WK, WV)
    kvh2 = kvh3.reshape(2 * N, D)
    rowmap = jnp.asarray(_ROWMAP)
    src2d = jnp.concatenate(
        [edge_index[0].reshape(EROWS, IW), jnp.zeros((1, IW), jnp.int32)], axis=0
    )
    dst2d = jnp.concatenate(
        [edge_index[1].reshape(EROWS, IW), jnp.zeros((1, IW), jnp.int32)], axis=0
    )
    src_slabs = jnp.take(src2d, rowmap, axis=0)
    dst_slabs = jnp.take(dst2d, rowmap, axis=0)
    srcg = jnp.concatenate([src_slabs, src_slabs + N], axis=0)
    acc2 = _edge_sc(srcg, dst_slabs, q, kvh2)
    h1, stats = _attnout(acc2, h, WO, bO)
    h3, stats2 = _bn_ffn(h1, stats, bn1_g, bn1_b, W1, b1, W2, b2)
    return _bn2(h3, stats2, bn2_g, bn2_b)


# speedup vs baseline: 9.1262x; 1.0003x over previous
import functools

import jax
import jax.numpy as jnp
import numpy as np
from jax import lax
from jax.experimental import pallas as pl
from jax.experimental.pallas import tpu as pltpu
from jax.experimental.pallas import tpu_sc as plsc

N = 10000
E = 320000
D = 128
H = 8
DH = 16
HC = H // 2
ACC_W = 72

C = 64
IW = 128
EROWS = E // IW
SLAB = 160
RPT = 1000

ROWS = 1000
GRID = N // ROWS

_NCH = [157 if t < 4 else 156 for t in range(16)]
_RS = [157 * t if t < 4 else 628 + 156 * (t - 4) for t in range(16)]
_ROWMAP = np.array(
    [[_RS[t] + r if r < _NCH[t] else EROWS for r in range(SLAB)] for t in range(16)],
    dtype=np.int32,
)


def _qkv_body(h_ref, wq_ref, wk_ref, wv_ref, q_ref, kvh_ref):
    x = h_ref[...]
    q = jnp.dot(x, wq_ref[...], preferred_element_type=jnp.float32)
    k = jnp.dot(x, wk_ref[...], preferred_element_type=jnp.float32)
    v = jnp.dot(x, wv_ref[...], preferred_element_type=jnp.float32)
    half = HC * DH
    q_ref[...] = q
    kvh_ref[0] = jnp.concatenate([k[:, :half], v[:, :half]], axis=1)
    kvh_ref[1] = jnp.concatenate([k[:, half:], v[:, half:]], axis=1)


def _qkv(h, WQ, WK, WV):
    return pl.pallas_call(
        _qkv_body,
        grid=(GRID,),
        in_specs=[
            pl.BlockSpec((ROWS, D), lambda i: (i, 0)),
            pl.BlockSpec((D, D), lambda i: (0, 0)),
            pl.BlockSpec((D, D), lambda i: (0, 0)),
            pl.BlockSpec((D, D), lambda i: (0, 0)),
        ],
        out_specs=[
            pl.BlockSpec((ROWS, D), lambda i: (i, 0)),
            pl.BlockSpec((2, ROWS, D), lambda i: (0, i, 0)),
        ],
        out_shape=[
            jax.ShapeDtypeStruct((N, D), jnp.float32),
            jax.ShapeDtypeStruct((2, N, D), jnp.float32),
        ],
    )(h, WQ, WK, WV)


def _edge_body(srcg_ref, dst_ref, q_ref, kvh_ref, out_ref,
               src_t, dst_t, dstl_row, kv0, kv1, q0, q1, out_b, acc_sh,
               sem_kv0, sem_kv1, sem_q0, sem_q1):
    cid = lax.axis_index("c")
    sid = lax.axis_index("s")

    nch = jnp.where(sid < 4, 2 * 157, 2 * 156)

    pltpu.sync_copy(srcg_ref.at[cid * 16 + sid], src_t)
    pltpu.sync_copy(dst_ref.at[sid], dst_t)

    @pl.loop(0, C)
    def _(ee):
        for j in range(4):
            out_b[ee, pl.ds(j * 16, 16)] = jnp.zeros((16,), jnp.float32)
        out_b[ee, pl.ds(ACC_W - 16, 16)] = jnp.zeros((16,), jnp.float32)

    @pl.when(sid < N // RPT)
    def _():
        @pl.loop(0, RPT // 40)
        def _(j):
            pltpu.sync_copy(out_b.at[pl.ds(0, 40)],
                            acc_sh.at[pl.ds(sid * RPT + j * 40, 40)])

    plsc.subcore_barrier()

    kvbufs = (kv0, kv1)
    qbufs = (q0, q1)
    semkv = (sem_kv0, sem_kv1)
    semq = (sem_q0, sem_q1)

    def issue(g, b):
        r = g // 2
        hf = (g % 2) * C
        pltpu.async_copy(kvh_ref.at[src_t.at[r, pl.ds(hf, C)]], kvbufs[b], semkv[b])
        pltpu.async_copy(q_ref.at[dst_t.at[r, pl.ds(hf, C)]], qbufs[b], semq[b])

    def wait(b):
        pltpu.make_async_copy(kvh_ref.at[src_t.at[0, pl.ds(0, C)]], kvbufs[b], semkv[b]).wait()
        pltpu.make_async_copy(q_ref.at[dst_t.at[0, pl.ds(0, C)]], qbufs[b], semq[b]).wait()

    def compute(g, b):
        kvb = kvbufs[b]
        qb = qbufs[b]

        @pl.loop(0, C // 16)
        def _(e16):
            lane = lax.broadcasted_iota(jnp.int32, (16,), 0)
            eidx = e16 * 16 + lane
            rots = [(lane + t) & 15 for t in range(DH)]
            qoff = cid * (HC * DH)
            for hh in range(HC):
                base = hh * DH
                colvs = [rots[t] + base for t in range(DH)]
                a0 = jnp.zeros((16,), jnp.float32)
                a1 = jnp.zeros((16,), jnp.float32)
                a2 = jnp.zeros((16,), jnp.float32)
                a3 = jnp.zeros((16,), jnp.float32)
                for t in range(DH):
                    colv = colvs[t]
                    kvals = plsc.load_gather(kvb, [eidx, colv])
                    qvals = plsc.load_gather(qb, [eidx, colv + qoff])
                    prod = kvals * qvals
                    if t % 4 == 0:
                        a0 = a0 + prod
                    elif t % 4 == 1:
                        a1 = a1 + prod
                    elif t % 4 == 2:
                        a2 = a2 + prod
                    else:
                        a3 = a3 + prod
                s = (a0 + a1) + (a2 + a3)
                w = jnp.exp(jnp.clip(s * 0.25, -5.0, 5.0))
                plsc.store_scatter(out_b, [eidx, jnp.full((16,), HC * DH + hh, jnp.int32)], w)
                for t in range(DH):
                    colv = colvs[t]
                    vvals = plsc.load_gather(kvb, [eidx, colv + (HC * DH)])
                    plsc.store_scatter(out_b, [eidx, colv], w * vvals)

        r = g // 2
        hf = (g % 2) * C

        @pl.loop(0, C // 16)
        def _(jj):
            dstl_row[pl.ds(jj * 16, 16)] = dst_t[r, pl.ds(hf + jj * 16, 16)]

        pltpu.sync_copy(out_b, acc_sh.at[dstl_row], add=True)

    issue(0, 0)

    @pl.loop(0, 314, step=2)
    def _(g):
        for b in range(2):
            gi = g + b

            @pl.when(gi < nch)
            def _():
                @pl.when(gi + 1 < nch)
                def _():
                    issue(gi + 1, 1 - b)

                wait(b)
                compute(gi, b)

    plsc.subcore_barrier()

    @pl.when(sid < N // RPT)
    def _():
        @pl.loop(0, RPT // 40)
        def _(j):
            pltpu.sync_copy(acc_sh.at[pl.ds(sid * RPT + j * 40, 40)],
                            out_ref.at[pl.ds(cid * N + sid * RPT + j * 40, 40),
                                       pl.ds(0, ACC_W)])


@functools.partial(
    pl.kernel,
    out_type=jax.ShapeDtypeStruct((2 * N, D), jnp.float32),
    mesh=plsc.VectorSubcoreMesh(core_axis_name="c", subcore_axis_name="s"),
    compiler_params=pltpu.CompilerParams(use_tc_tiling_on_sc=False, needs_layout_passes=False),
    scratch_types=[
        pltpu.VMEM((SLAB, IW), jnp.int32),
        pltpu.VMEM((SLAB, IW), jnp.int32),
        pltpu.VMEM((C,), jnp.int32),
        pltpu.VMEM((C, D), jnp.float32),
        pltpu.VMEM((C, D), jnp.float32),
        pltpu.VMEM((C, D), jnp.float32),
        pltpu.VMEM((C, D), jnp.float32),
        pltpu.VMEM((C, ACC_W), jnp.float32),
        pltpu.VMEM_SHARED((N, ACC_W), jnp.float32),
        pltpu.SemaphoreType.DMA,
        pltpu.SemaphoreType.DMA,
        pltpu.SemaphoreType.DMA,
        pltpu.SemaphoreType.DMA,
    ],
)
def _edge_sc(srcg_ref, dst_ref, q_ref, kvh_ref, out_ref,
             src_t, dst_t, dstl_row, kv0, kv1, q0, q1, out_b, acc_sh,
             sem_kv0, sem_kv1, sem_q0, sem_q1):
    _edge_body(srcg_ref, dst_ref, q_ref, kvh_ref, out_ref,
               src_t, dst_t, dstl_row, kv0, kv1, q0, q1, out_b, acc_sh,
               sem_kv0, sem_kv1, sem_q0, sem_q1)


def _attnout_body(acca_ref, accb_ref, h_ref, wo_ref, bo_ref, h1_ref, stats_ref):
    acca = acca_ref[...]
    accb = accb_ref[...]
    zcol = HC * DH
    z = jnp.concatenate([acca[:, zcol:zcol + HC], accb[:, zcol:zcol + HC]], axis=1)
    zr = 1.0 / (z + 1e-6)
    hsel = lax.broadcasted_iota(jnp.int32, (H, D), 0)
    dsel = lax.broadcasted_iota(jnp.int32, (H, D), 1) // DH
    expand = (hsel == dsel).astype(jnp.float32)
    zfull = jnp.dot(zr, expand, preferred_element_type=jnp.float32)
    accv = jnp.concatenate([acca[:, :zcol], accb[:, :zcol]], axis=1)
    h_attn = accv * zfull
    h_out = jnp.dot(h_attn, wo_ref[...], preferred_element_type=jnp.float32)
    h1 = h_ref[...] + h_out + bo_ref[...]
    h1_ref[...] = h1
    s = jnp.concatenate(
        [jnp.sum(h1, axis=0, keepdims=True), jnp.sum(h1 * h1, axis=0, keepdims=True)], axis=0
    )

    @pl.when(pl.program_id(0) == 0)
    def _():
        stats_ref[...] = jnp.zeros_like(stats_ref)

    stats_ref[...] += s


def _attnout(acc2, h, WO, bO):
    return pl.pallas_call(
        _attnout_body,
        grid=(GRID,),
        in_specs=[
            pl.BlockSpec((ROWS, D), lambda i: (i, 0)),
            pl.BlockSpec((ROWS, D), lambda i: (GRID + i, 0)),
            pl.BlockSpec((ROWS, D), lambda i: (i, 0)),
            pl.BlockSpec((D, D), lambda i: (0, 0)),
            pl.BlockSpec((1, D), lambda i: (0, 0)),
        ],
        out_specs=[
            pl.BlockSpec((ROWS, D), lambda i: (i, 0)),
            pl.BlockSpec((2, D), lambda i: (0, 0)),
        ],
        out_shape=[
            jax.ShapeDtypeStruct((N, D), jnp.float32),
            jax.ShapeDtypeStruct((2, D), jnp.float32),
        ],
    )(acc2, acc2, h, WO, bO.reshape(1, D))


def _bn_ffn_body(h1_ref, stats_ref, g_ref, b_ref, w1_ref, b1_ref, w2_ref, b2_ref,
                 h3_ref, stats2_ref):
    s = stats_ref[...]
    mu = s[0:1, :] / N
    var = s[1:2, :] / N - mu * mu
    scale = g_ref[...] * lax.rsqrt(var + 1e-5)
    hn = (h1_ref[...] - mu) * scale + b_ref[...]
    a = jnp.dot(hn, w1_ref[...], preferred_element_type=jnp.float32) + b1_ref[...]
    a = jnp.maximum(a, 0.0)
    h2 = jnp.dot(a, w2_ref[...], preferred_element_type=jnp.float32) + b2_ref[...]
    h3 = hn + h2
    h3_ref[...] = h3
    s2 = jnp.concatenate(
        [jnp.sum(h3, axis=0, keepdims=True), jnp.sum(h3 * h3, axis=0, keepdims=True)], axis=0
    )

    @pl.when(pl.program_id(0) == 0)
    def _():
        stats2_ref[...] = jnp.zeros_like(stats2_ref)

    stats2_ref[...] += s2


def _bn_ffn(h1, stats, bn1_g, bn1_b, W1, b1, W2, b2):
    return pl.pallas_call(
        _bn_ffn_body,
        grid=(GRID,),
        in_specs=[
            pl.BlockSpec((ROWS, D), lambda i: (i, 0)),
            pl.BlockSpec((2, D), lambda i: (0, 0)),
            pl.BlockSpec((1, D), lambda i: (0, 0)),
            pl.BlockSpec((1, D), lambda i: (0, 0)),
            pl.BlockSpec((D, 2 * D), lambda i: (0, 0)),
            pl.BlockSpec((1, 2 * D), lambda i: (0, 0)),
            pl.BlockSpec((2 * D, D), lambda i: (0, 0)),
            pl.BlockSpec((1, D), lambda i: (0, 0)),
        ],
        out_specs=[
            pl.BlockSpec((ROWS, D), lambda i: (i, 0)),
            pl.BlockSpec((2, D), lambda i: (0, 0)),
        ],
        out_shape=[
            jax.ShapeDtypeStruct((N, D), jnp.float32),
            jax.ShapeDtypeStruct((2, D), jnp.float32),
        ],
    )(h1, stats, bn1_g.reshape(1, D), bn1_b.reshape(1, D), W1, b1.reshape(1, 2 * D),
      W2, b2.reshape(1, D))


def _bn2_body(h3_ref, stats_ref, g_ref, b_ref, out_ref):
    s = stats_ref[...]
    mu = s[0:1, :] / N
    var = s[1:2, :] / N - mu * mu
    scale = g_ref[...] * lax.rsqrt(var + 1e-5)
    out_ref[...] = (h3_ref[...] - mu) * scale + b_ref[...]


def _bn2(h3, stats2, bn2_g, bn2_b):
    return pl.pallas_call(
        _bn2_body,
        grid=(GRID,),
        in_specs=[
            pl.BlockSpec((ROWS, D), lambda i: (i, 0)),
            pl.BlockSpec((2, D), lambda i: (0, 0)),
            pl.BlockSpec((1, D), lambda i: (0, 0)),
            pl.BlockSpec((1, D), lambda i: (0, 0)),
        ],
        out_specs=pl.BlockSpec((ROWS, D), lambda i: (i, 0)),
        out_shape=jax.ShapeDtypeStruct((N, D), jnp.float32),
    )(h3, stats2, bn2_g.reshape(1, D), bn2_b.reshape(1, D))


def kernel(h, p, e, edge_index, WQ, WK, WE, WV, WO, bO, bn1_g, bn1_b, W1, b1, W2, b2,
           bn2_g, bn2_b):
    q, kvh3 = _qkv(h, WQ, WK, WV)
    kvh2 = kvh3.reshape(2 * N, D)
    rowmap = jnp.asarray(_ROWMAP)
    src2d = jnp.concatenate(
        [edge_index[0].reshape(EROWS, IW), jnp.zeros((1, IW), jnp.int32)], axis=0
    )
    dst2d = jnp.concatenate(
        [edge_index[1].reshape(EROWS, IW), jnp.zeros((1, IW), jnp.int32)], axis=0
    )
    src_slabs = jnp.take(src2d, rowmap, axis=0)
    dst_slabs = jnp.take(dst2d, rowmap, axis=0)
    srcg = jnp.concatenate([src_slabs, src_slabs + N], axis=0)
    acc2 = _edge_sc(srcg, dst_slabs, q, kvh2)
    h1, stats = _attnout(acc2, h, WO, bO)
    h3, stats2 = _bn_ffn(h1, stats, bn1_g, bn1_b, W1, b1, W2, b2)
    return _bn2(h3, stats2, bn2_g, bn2_b)
